# baseline (speedup 1.0000x reference)
import functools

import jax
import jax.numpy as jnp
from jax import lax
from jax.experimental import pallas as pl
from jax.experimental.pallas import tpu as pltpu
from jax.experimental.pallas import tpu_sc as plsc

_BETA = 1.0 / 9.0
_LN2 = 0.6931471805599453
_SQRT2 = 1.4142135623730951

_B, _N, _C = 2, 126720, 4
_A = _B * _N
_NW = 32
_APW = _A // _NW
_CH = 880
_NSTEP = _APW // _CH


def _smooth_l1(diff):
    ad = jnp.abs(diff)
    t = jnp.minimum(ad, _BETA)
    return ad - t + t * t * (0.5 / _BETA)


def _ln(x):
    xi = plsc.bitcast(x, jnp.int32)
    e = (xi >> 23) - 127
    m = plsc.bitcast((xi & 0x007FFFFF) | 0x3F800000, jnp.float32)
    big = m > _SQRT2
    m = jnp.where(big, m * 0.5, m)
    ef = e.astype(jnp.float32) + jnp.where(big, 1.0, 0.0)
    z = (m - 1.0) / (m + 1.0)
    z2 = z * z
    p = 2.0 * z * (1.0 + z2 * (1.0 / 3.0 + z2 * (1.0 / 5.0 + z2 * (1.0 / 7.0))))
    return ef * _LN2 + p


def _sc_body(cls_h, b2d_h, b3d_h, b2dt_h, b3dt_h, w_h, lab_h, out_h,
             cls_v, b2d_v, b2dt_v, b3d_v, b3dt_v, w_v, lab_v, weff_v, obuf):
    wid = lax.axis_index("s") * 2 + lax.axis_index("c")
    a0 = wid * _APW
    lane = lax.iota(jnp.int32, 16)
    zero = jnp.zeros((16,), jnp.float32)

    def step(s, accs):
        s3a, s2a, cea, fga, acta = accs
        base = a0 + s * _CH
        pltpu.sync_copy(cls_h.at[pl.ds(base * 4, _CH * 4)], cls_v)
        pltpu.sync_copy(b2d_h.at[pl.ds(base * 4, _CH * 4)], b2d_v)
        pltpu.sync_copy(b2dt_h.at[pl.ds(base * 4, _CH * 4)], b2dt_v)
        pltpu.sync_copy(b3d_h.at[pl.ds(base * 11, _CH * 11)], b3d_v)
        pltpu.sync_copy(b3dt_h.at[pl.ds(base * 11, _CH * 11)], b3dt_v)
        pltpu.sync_copy(w_h.at[pl.ds(base, _CH)], w_v)
        pltpu.sync_copy(lab_h.at[pl.ds(base, _CH)], lab_v)

        def wbody(i, c):
            fga, acta = c
            lb = lab_v[pl.ds(i * 16, 16)]
            fg1 = jnp.where(lb > 0, 1.0, 0.0)
            weff_v[pl.ds(i * 16, 16)] = fg1 * w_v[pl.ds(i * 16, 16)]
            return (fga + fg1, acta + jnp.where(lb >= 0, 1.0, 0.0))

        fga, acta = lax.fori_loop(0, _CH // 16, wbody, (fga, acta))

        def cbody(g, cea):
            ai4 = (g * 16 + lane) * 4
            x0 = plsc.load_gather(cls_v, [ai4])
            x1 = plsc.load_gather(cls_v, [ai4 + 1])
            x2 = plsc.load_gather(cls_v, [ai4 + 2])
            x3 = plsc.load_gather(cls_v, [ai4 + 3])
            se = jnp.exp(x0) + jnp.exp(x1) + jnp.exp(x2) + jnp.exp(x3)
            lb = lab_v[pl.ds(g * 16, 16)]
            sel = jnp.where(lb == 0, x0,
                            jnp.where(lb == 1, x1,
                                      jnp.where(lb == 2, x2, x3)))
            act = jnp.where(lb >= 0, 1.0, 0.0)
            return cea + (_ln(se) - sel) * act

        cea = lax.fori_loop(0, _CH // 16, cbody, cea)

        def b2body(v, s2a):
            d = b2d_v[pl.ds(v * 16, 16)] - b2dt_v[pl.ds(v * 16, 16)]
            wg = plsc.load_gather(weff_v, [(v * 16 + lane) >> 2])
            return s2a + _smooth_l1(d) * wg

        s2a = lax.fori_loop(0, _CH * 4 // 16, b2body, s2a)

        def b3body(v, s3a):
            d = b3d_v[pl.ds(v * 16, 16)] - b3dt_v[pl.ds(v * 16, 16)]
            wg = plsc.load_gather(weff_v, [(v * 16 + lane) // 11])
            return s3a + _smooth_l1(d) * wg

        s3a = lax.fori_loop(0, _CH * 11 // 16, b3body, s3a)
        return (s3a, s2a, cea, fga, acta)

    s3a, s2a, cea, fga, acta = lax.fori_loop(
        0, _NSTEP, step, (zero, zero, zero, zero, zero))

    obuf[pl.ds(0, 16)] = s3a
    obuf[pl.ds(16, 16)] = s2a
    obuf[pl.ds(32, 16)] = cea
    obuf[pl.ds(48, 16)] = fga
    obuf[pl.ds(64, 16)] = acta
    obuf[pl.ds(80, 16)] = zero
    obuf[pl.ds(96, 16)] = zero
    obuf[pl.ds(112, 16)] = zero
    pltpu.sync_copy(obuf, out_h.at[wid])


@functools.lru_cache(maxsize=1)
def _sc_loss():
    return functools.partial(
        pl.kernel,
        mesh=plsc.VectorSubcoreMesh(core_axis_name="c", subcore_axis_name="s"),
        out_type=jax.ShapeDtypeStruct((_NW, 128), jnp.float32),
        compiler_params=pltpu.CompilerParams(needs_layout_passes=False),
        scratch_types=[
            pltpu.VMEM((_CH * 4,), jnp.float32),
            pltpu.VMEM((_CH * 4,), jnp.float32),
            pltpu.VMEM((_CH * 4,), jnp.float32),
            pltpu.VMEM((_CH * 11,), jnp.float32),
            pltpu.VMEM((_CH * 11,), jnp.float32),
            pltpu.VMEM((_CH,), jnp.float32),
            pltpu.VMEM((_CH,), jnp.int32),
            pltpu.VMEM((_CH,), jnp.float32),
            pltpu.VMEM((128,), jnp.float32),
        ],
    )(_sc_body)


@jax.jit
def kernel(cls, bbox_2d, bbox_3d, bbox_2d_tar, bbox_3d_tar, bbox_weights,
           labels):
    part = _sc_loss()(
        cls.reshape(-1),
        bbox_2d.reshape(-1),
        bbox_3d.reshape(-1),
        bbox_2d_tar.reshape(-1),
        bbox_3d_tar.reshape(-1),
        bbox_weights.reshape(-1),
        labels.reshape(-1).astype(jnp.int32),
    )
    s3 = jnp.sum(part[:, 0:16])
    s2 = jnp.sum(part[:, 16:32])
    ce = jnp.sum(part[:, 32:48])
    fg = jnp.sum(part[:, 48:64])
    act = jnp.sum(part[:, 64:80])
    return ce / jnp.maximum(act, 1.0) + (s2 + s3) / jnp.maximum(fg, 1.0)


# speedup vs baseline: 13.1173x; 13.1173x over previous
import functools

import jax
import jax.numpy as jnp
from jax import lax
from jax.experimental import pallas as pl
from jax.experimental.pallas import tpu as pltpu
from jax.experimental.pallas import tpu_sc as plsc

_BETA = 1.0 / 9.0
_LN2 = 0.6931471805599453
_SQRT2 = 1.4142135623730951

_B, _N, _C = 2, 126720, 4
_T = _N // 128
_NW = 32
_S = 5
_NSTEP = 6


def _smooth_l1(diff):
    ad = jnp.abs(diff)
    t = jnp.minimum(ad, _BETA)
    return ad - t + t * t * (0.5 / _BETA)


def _ln(x):
    xi = plsc.bitcast(x, jnp.int32)
    e = (xi >> 23) - 127
    m = plsc.bitcast((xi & 0x007FFFFF) | 0x3F800000, jnp.float32)
    big = m > _SQRT2
    m = jnp.where(big, m * 0.5, m)
    ef = e.astype(jnp.float32) + jnp.where(big, 1.0, 0.0)
    z = (m - 1.0) / (m + 1.0)
    z2 = z * z
    p = 2.0 * z * (1.0 + z2 * (1.0 / 3.0 + z2 * (1.0 / 5.0 + z2 * (1.0 / 7.0))))
    return ef * _LN2 + p


def _sc_body(cls_h, b2d_h, b3d_h, b2dt_h, b3dt_h, w_h, lab_h, out_h,
             cls_v, b2d_v, b2dt_v, b3d_v, b3dt_v, w_v, lab_v, obuf, sem):
    wid = lax.axis_index("s") * 2 + lax.axis_index("c")
    t0 = jnp.where(wid < 30, wid * 31, 930 + (wid - 30) * 30)
    zero = jnp.zeros((16,), jnp.float32)

    def stage(tbase, s_chunks):
        hs = [
            pltpu.async_copy(b3d_h.at[:, pl.ds(tbase, s_chunks)],
                             b3d_v.at[:, pl.ds(0, s_chunks)], sem),
            pltpu.async_copy(b3dt_h.at[:, pl.ds(tbase, s_chunks)],
                             b3dt_v.at[:, pl.ds(0, s_chunks)], sem),
            pltpu.async_copy(cls_h.at[:, pl.ds(tbase, s_chunks)],
                             cls_v.at[:, pl.ds(0, s_chunks)], sem),
            pltpu.async_copy(b2d_h.at[:, pl.ds(tbase, s_chunks)],
                             b2d_v.at[:, pl.ds(0, s_chunks)], sem),
            pltpu.async_copy(b2dt_h.at[:, pl.ds(tbase, s_chunks)],
                             b2dt_v.at[:, pl.ds(0, s_chunks)], sem),
            pltpu.async_copy(w_h.at[pl.ds(tbase, s_chunks)],
                             w_v.at[pl.ds(0, s_chunks)], sem),
            pltpu.async_copy(lab_h.at[pl.ds(tbase, s_chunks)],
                             lab_v.at[pl.ds(0, s_chunks)], sem),
        ]
        for h in hs:
            h.wait()

    def compute(s_chunks, accs):
        def body(i, accs):
            s3a, s2a, cea, fga, acta = accs
            tt = i >> 4
            b = (i >> 3) & 1
            j = i & 7
            ds = pl.ds(j * 16, 16)

            lb = lab_v[tt, b, ds]
            fg1 = jnp.where(lb > 0, 1.0, 0.0)
            act = jnp.where(lb >= 0, 1.0, 0.0)
            weff = fg1 * w_v[tt, b, ds]

            x0 = cls_v[b, tt, 0, ds]
            x1 = cls_v[b, tt, 1, ds]
            x2 = cls_v[b, tt, 2, ds]
            x3 = cls_v[b, tt, 3, ds]
            se = jnp.exp(x0) + jnp.exp(x1) + jnp.exp(x2) + jnp.exp(x3)
            sel = jnp.where(lb == 0, x0,
                            jnp.where(lb == 1, x1,
                                      jnp.where(lb == 2, x2, x3)))
            cea = cea + (_ln(se) - sel) * act

            l2 = zero
            for c in range(4):
                d = b2d_v[b, tt, c, ds] - b2dt_v[b, tt, c, ds]
                l2 = l2 + _smooth_l1(d)
            l3 = zero
            for c in range(11):
                d = b3d_v[c, tt, b, ds] - b3dt_v[c, tt, b, ds]
                l3 = l3 + _smooth_l1(d)
            return (s3a + l3 * weff, s2a + l2 * weff, cea,
                    fga + fg1, acta + act)

        return lax.fori_loop(0, s_chunks * 16, body, accs)

    def step(s, accs):
        stage(t0 + s * _S, _S)
        return compute(_S, accs)

    accs = lax.fori_loop(0, _NSTEP, step,
                         (zero, zero, zero, zero, zero))

    tail_t = jnp.minimum(t0 + 30, _T - 1)
    stage(tail_t, 1)
    tp = compute(1, (zero, zero, zero, zero, zero))
    m = jnp.where(wid < 30, 1.0, 0.0)
    s3a = accs[0] + m * tp[0]
    s2a = accs[1] + m * tp[1]
    cea = accs[2] + m * tp[2]
    fga = accs[3] + m * tp[3]
    acta = accs[4] + m * tp[4]

    obuf[pl.ds(0, 16)] = s3a
    obuf[pl.ds(16, 16)] = s2a
    obuf[pl.ds(32, 16)] = cea
    obuf[pl.ds(48, 16)] = fga
    obuf[pl.ds(64, 16)] = acta
    obuf[pl.ds(80, 16)] = zero
    obuf[pl.ds(96, 16)] = zero
    obuf[pl.ds(112, 16)] = zero
    pltpu.sync_copy(obuf, out_h.at[wid])


@functools.lru_cache(maxsize=1)
def _sc_loss():
    return functools.partial(
        pl.kernel,
        mesh=plsc.VectorSubcoreMesh(core_axis_name="c", subcore_axis_name="s"),
        out_type=jax.ShapeDtypeStruct((_NW, 128), jnp.float32),
        compiler_params=pltpu.CompilerParams(needs_layout_passes=False),
        scratch_types=[
            pltpu.VMEM((2, _S, 4, 128), jnp.float32),
            pltpu.VMEM((2, _S, 4, 128), jnp.float32),
            pltpu.VMEM((2, _S, 4, 128), jnp.float32),
            pltpu.VMEM((11, _S, 2, 128), jnp.float32),
            pltpu.VMEM((11, _S, 2, 128), jnp.float32),
            pltpu.VMEM((_S, 2, 128), jnp.float32),
            pltpu.VMEM((_S, 2, 128), jnp.int32),
            pltpu.VMEM((128,), jnp.float32),
            pltpu.SemaphoreType.DMA,
        ],
    )(_sc_body)


def _coord_major(x, c):
    return (x.transpose(2, 0, 1).reshape(c, 2, _T, 128)
            .transpose(0, 2, 1, 3))


def _batch_major(x, c):
    return (x.transpose(0, 2, 1).reshape(2, c, _T, 128)
            .transpose(0, 2, 1, 3))


def _chunk_major(x):
    return x.reshape(2, _T, 128).transpose(1, 0, 2)


@jax.jit
def kernel(cls, bbox_2d, bbox_3d, bbox_2d_tar, bbox_3d_tar, bbox_weights,
           labels):
    part = _sc_loss()(
        _batch_major(cls, 4),
        _batch_major(bbox_2d, 4),
        _coord_major(bbox_3d, 11),
        _batch_major(bbox_2d_tar, 4),
        _coord_major(bbox_3d_tar, 11),
        _chunk_major(bbox_weights),
        _chunk_major(labels).astype(jnp.int32),
    )
    s3 = jnp.sum(part[:, 0:16])
    s2 = jnp.sum(part[:, 16:32])
    ce = jnp.sum(part[:, 32:48])
    fg = jnp.sum(part[:, 48:64])
    act = jnp.sum(part[:, 64:80])
    return ce / jnp.maximum(act, 1.0) + (s2 + s3) / jnp.maximum(fg, 1.0)
